# baseline (speedup 1.0000x reference)
import functools

import jax
import jax.numpy as jnp
from jax import lax
from jax.experimental import pallas as pl
from jax.experimental.pallas import tpu as pltpu
from jax.experimental.pallas import tpu_sc as plsc

N = 10000
NP = 10240
E = 320000
IN_D = 128
HID = 128
OUT_D = 64

C = 128
NCORES = 2
NSUB = 16
NW = NCORES * NSUB
CPT = 80
PADE = NW * CPT * C
NCHUNK = PADE // C
INIT_ROWS = N // NSUB
DREP = 16
EPS = 1e-5

RB = 2000
GRID = N // RB


def _sc_mesh():
    return plsc.VectorSubcoreMesh(core_axis_name="c", subcore_axis_name="s")


@functools.partial(
    pl.kernel,
    out_type=jax.ShapeDtypeStruct((NCORES, N, DREP), jnp.float32),
    mesh=_sc_mesh(),
    compiler_params=pltpu.CompilerParams(use_tc_tiling_on_sc=False),
    scratch_types=[
        pltpu.VMEM((CPT, 2, C), jnp.int32),
        pltpu.VMEM((C, DREP), jnp.float32),
        pltpu.VMEM_SHARED((NP, DREP), jnp.float32),
    ],
)
def _deg_kernel(edge_hbm, ones_hbm, zero_hbm, out_hbm, idx, ones_v, acc):
    c = lax.axis_index("c")
    s = lax.axis_index("s")
    w = c * NSUB + s
    r0 = s * INIT_ROWS
    pltpu.sync_copy(zero_hbm.at[pl.ds(r0, INIT_ROWS)],
                    acc.at[pl.ds(r0, INIT_ROWS)])
    tr = (NP - N) // NSUB

    @pl.when(s == 0)
    def _():
        pltpu.sync_copy(zero_hbm.at[pl.ds(0, NP - N)],
                        acc.at[pl.ds(N, NP - N)])

    del tr
    pltpu.sync_copy(ones_hbm, ones_v)
    pltpu.sync_copy(edge_hbm.at[pl.ds(w * CPT, CPT)], idx)
    plsc.subcore_barrier()

    def body(i, carry):
        pltpu.sync_copy(ones_v, acc.at[idx.at[i, 1]], add=True)
        return carry

    lax.fori_loop(0, CPT, body, 0)
    plsc.subcore_barrier()
    pltpu.sync_copy(acc.at[pl.ds(r0, INIT_ROWS)],
                    out_hbm.at[c, pl.ds(r0, INIT_ROWS)])


def _make_scatter(D, NB):
    NG = CPT // NB

    @functools.partial(
        pl.kernel,
        out_type=jax.ShapeDtypeStruct((NCORES, N, D), jnp.float32),
        mesh=_sc_mesh(),
        compiler_params=pltpu.CompilerParams(use_tc_tiling_on_sc=False),
        scratch_types=[
            pltpu.VMEM((2, NB, 2, C), jnp.int32),
            pltpu.VMEM((NB, C, D), jnp.float32),
            pltpu.VMEM_SHARED((NP, D), jnp.float32),
        ] + [pltpu.SemaphoreType.DMA] * (NB + 1),
    )
    def _scatter(h_hbm, edge_hbm, out_hbm, idx, rows, acc, *sems):
        gsem = sems[:NB]
        isem = sems[NB]
        c = lax.axis_index("c")
        s = lax.axis_index("s")
        w = c * NSUB + s
        r0 = s * INIT_ROWS
        row0 = w * CPT
        pltpu.sync_copy(h_hbm.at[pl.ds(r0, INIT_ROWS)],
                        acc.at[pl.ds(r0, INIT_ROWS)])
        pltpu.sync_copy(edge_hbm.at[pl.ds(row0, NB)], idx.at[0])
        plsc.subcore_barrier()
        for b in range(NB):
            pltpu.async_copy(h_hbm.at[idx.at[0, b, 0]], rows.at[b], gsem[b])

        def group(g, carry):
            par = lax.rem(g, 2)
            nxt = 1 - par

            @pl.when(g < NG - 1)
            def _():
                off = row0 + (g + 1) * NB
                pltpu.async_copy(edge_hbm.at[pl.ds(off, NB)], idx.at[nxt],
                                 isem)

            for b in range(NB):
                pltpu.make_async_copy(h_hbm.at[idx.at[par, b, 0]],
                                      rows.at[b], gsem[b]).wait()
                pltpu.sync_copy(rows.at[b], acc.at[idx.at[par, b, 1]],
                                add=True)

            @pl.when(g < NG - 1)
            def _():
                pltpu.make_async_copy(edge_hbm.at[pl.ds(row0, NB)],
                                      idx.at[nxt], isem).wait()
                for b in range(NB):
                    pltpu.async_copy(h_hbm.at[idx.at[nxt, b, 0]], rows.at[b],
                                     gsem[b])
            return carry

        lax.fori_loop(0, NG, group, 0)
        plsc.subcore_barrier()
        pltpu.sync_copy(acc.at[pl.ds(r0, INIT_ROWS)],
                        out_hbm.at[c, pl.ds(r0, INIT_ROWS)])

    return _scatter


_scatter128 = _make_scatter(HID, 2)
_scatter64 = _make_scatter(OUT_D, 8)


def _t1_body(deg_ref, x_ref, w_ref, h_ref, dinv_ref):
    deg = deg_ref[0] + deg_ref[1] + 1.0
    dinv = lax.rsqrt(deg)
    dr = jnp.broadcast_to(dinv[:, 0:1], (RB, HID))
    h = jnp.dot(x_ref[...], w_ref[...], preferred_element_type=jnp.float32)
    h_ref[...] = h * dr
    dinv_ref[...] = dr


def _t_mid_body(p_ref, h_ref, dinv_ref, b_ref, g_ref, be_ref, w_ref, o_ref):
    s = p_ref[0] + p_ref[1] - h_ref[...]
    o = s * dinv_ref[...] + b_ref[...]
    mu = jnp.mean(o, axis=1, keepdims=True)
    d = o - mu
    var = jnp.mean(d * d, axis=1, keepdims=True)
    ln = d * lax.rsqrt(var + EPS) * g_ref[...] + be_ref[...]
    r = jnp.maximum(ln, 0.0)
    o_ref[...] = (jnp.dot(r, w_ref[...], preferred_element_type=jnp.float32)
                  * dinv_ref[:, 0:1])


def _t4_body(p_ref, h_ref, dinv_ref, b_ref, o_ref):
    s = p_ref[0] + p_ref[1] - h_ref[...]
    o_ref[...] = s * dinv_ref[:, 0:1] + b_ref[...]


def _row_spec(d):
    return pl.BlockSpec((RB, d), lambda i: (i, 0))


def _pair_spec(d):
    return pl.BlockSpec((NCORES, RB, d), lambda i: (0, i, 0))


def _const_spec(shape):
    nd = len(shape)
    return pl.BlockSpec(shape, lambda i: (0,) * nd)


def kernel(x, edge_index, W1, b1, W2, b2, W3, b3, gamma, beta):
    src = edge_index[0].astype(jnp.int32)
    dst = edge_index[1].astype(jnp.int32)
    pad = PADE - E
    pad_src = (jnp.arange(pad, dtype=jnp.int32) * 37) % N
    src_p = jnp.concatenate([src, pad_src])
    pad_dst = N + (jnp.arange(pad, dtype=jnp.int32) % (NP - N))
    dst_p = jnp.concatenate([dst, pad_dst])
    edge_p = jnp.stack([src_p.reshape(NCHUNK, C), dst_p.reshape(NCHUNK, C)],
                       axis=1)
    ones_c = jnp.ones((C, DREP), jnp.float32)
    zeros_n = jnp.zeros((N, DREP), jnp.float32)
    b1r = b1.reshape(1, HID)
    b2r = b2.reshape(1, HID)
    b3r = b3.reshape(1, OUT_D)
    gr = gamma.reshape(1, HID)
    ber = beta.reshape(1, HID)

    deg_p = _deg_kernel(edge_p, ones_c, zeros_n)

    h1, dinv = pl.pallas_call(
        _t1_body,
        grid=(GRID,),
        in_specs=[_pair_spec(DREP), _row_spec(IN_D), _const_spec((IN_D, HID))],
        out_specs=[_row_spec(HID), _row_spec(HID)],
        out_shape=[jax.ShapeDtypeStruct((N, HID), jnp.float32),
                   jax.ShapeDtypeStruct((N, HID), jnp.float32)],
    )(deg_p, x, W1)

    p1 = _scatter128(h1, edge_p)

    h2 = pl.pallas_call(
        _t_mid_body,
        grid=(GRID,),
        in_specs=[_pair_spec(HID), _row_spec(HID), _row_spec(HID),
                  _const_spec((1, HID)), _const_spec((1, HID)),
                  _const_spec((1, HID)), _const_spec((HID, HID))],
        out_specs=_row_spec(HID),
        out_shape=jax.ShapeDtypeStruct((N, HID), jnp.float32),
    )(p1, h1, dinv, b1r, gr, ber, W2)

    p2 = _scatter128(h2, edge_p)

    h3 = pl.pallas_call(
        _t_mid_body,
        grid=(GRID,),
        in_specs=[_pair_spec(HID), _row_spec(HID), _row_spec(HID),
                  _const_spec((1, HID)), _const_spec((1, HID)),
                  _const_spec((1, HID)), _const_spec((HID, OUT_D))],
        out_specs=_row_spec(OUT_D),
        out_shape=jax.ShapeDtypeStruct((N, OUT_D), jnp.float32),
    )(p2, h2, dinv, b2r, gr, ber, W3)

    p3 = _scatter64(h3, edge_p)

    out = pl.pallas_call(
        _t4_body,
        grid=(GRID,),
        in_specs=[_pair_spec(OUT_D), _row_spec(OUT_D), _row_spec(HID),
                  _const_spec((1, OUT_D))],
        out_specs=_row_spec(OUT_D),
        out_shape=jax.ShapeDtypeStruct((N, OUT_D), jnp.float32),
    )(p3, h3, dinv, b3r)

    return out


# speedup vs baseline: 1.1759x; 1.1759x over previous
import functools

import jax
import jax.numpy as jnp
from jax import lax
from jax.experimental import pallas as pl
from jax.experimental.pallas import tpu as pltpu
from jax.experimental.pallas import tpu_sc as plsc

N = 10000
NP = 10240
E = 320000
IN_D = 128
HID = 128
OUT_D = 64

C = 128
NCORES = 2
NSUB = 16
NW = NCORES * NSUB
CPT = 80
PADE = NW * CPT * C
NCHUNK = PADE // C
INIT_ROWS = N // NSUB
DREP = 16
EPS = 1e-5

RB = 2000
GRID = N // RB


def _sc_mesh():
    return plsc.VectorSubcoreMesh(core_axis_name="c", subcore_axis_name="s")


@functools.partial(
    pl.kernel,
    out_type=jax.ShapeDtypeStruct((NCORES, N, DREP), jnp.float32),
    mesh=_sc_mesh(),
    compiler_params=pltpu.CompilerParams(use_tc_tiling_on_sc=False),
    scratch_types=[
        pltpu.VMEM((CPT, C), jnp.int32),
        pltpu.VMEM((C, DREP), jnp.float32),
        pltpu.VMEM_SHARED((NP, DREP), jnp.float32),
    ],
)
def _deg_kernel(edge_hbm, ones_hbm, zero_hbm, out_hbm, idx, ones_v, acc):
    c = lax.axis_index("c")
    s = lax.axis_index("s")
    w = c * NSUB + s
    r0 = s * INIT_ROWS
    pltpu.sync_copy(zero_hbm.at[pl.ds(r0, INIT_ROWS)],
                    acc.at[pl.ds(r0, INIT_ROWS)])
    tr = (NP - N) // NSUB

    @pl.when(s == 0)
    def _():
        pltpu.sync_copy(zero_hbm.at[pl.ds(0, NP - N)],
                        acc.at[pl.ds(N, NP - N)])

    del tr
    pltpu.sync_copy(ones_hbm, ones_v)
    pltpu.sync_copy(edge_hbm.at[1, pl.ds(w * CPT, CPT)], idx)
    plsc.subcore_barrier()

    def body(i, carry):
        pltpu.sync_copy(ones_v, acc.at[idx.at[i]], add=True)
        return carry

    lax.fori_loop(0, CPT, body, 0)
    plsc.subcore_barrier()
    pltpu.sync_copy(acc.at[pl.ds(r0, INIT_ROWS)],
                    out_hbm.at[c, pl.ds(r0, INIT_ROWS)])


def _make_scatter(D, NB):
    NG = CPT // NB

    @functools.partial(
        pl.kernel,
        out_type=jax.ShapeDtypeStruct((NCORES, N, D), jnp.float32),
        mesh=_sc_mesh(),
        compiler_params=pltpu.CompilerParams(use_tc_tiling_on_sc=False),
        scratch_types=[
            pltpu.VMEM((2, NB, C), jnp.int32),
            pltpu.VMEM((NB, C), jnp.int32),
            pltpu.VMEM((NB, C, D), jnp.float32),
            pltpu.VMEM_SHARED((NP, D), jnp.float32),
        ] + [pltpu.SemaphoreType.DMA] * NB,
    )
    def _scatter(h_hbm, edge_hbm, out_hbm, sidx, didx, rows, acc, *gsem):
        c = lax.axis_index("c")
        s = lax.axis_index("s")
        w = c * NSUB + s
        r0 = s * INIT_ROWS
        row0 = w * CPT
        pltpu.sync_copy(h_hbm.at[pl.ds(r0, INIT_ROWS)],
                        acc.at[pl.ds(r0, INIT_ROWS)])
        pltpu.sync_copy(edge_hbm.at[0, pl.ds(row0, NB)], sidx.at[0])
        plsc.subcore_barrier()
        for b in range(NB):
            pltpu.async_copy(h_hbm.at[sidx.at[0, b]], rows.at[b], gsem[b])

        def group(g, carry):
            par = lax.rem(g, 2)
            nxt = 1 - par

            @pl.when(g < NG - 1)
            def _():
                pltpu.sync_copy(edge_hbm.at[0, pl.ds(row0 + (g + 1) * NB, NB)],
                                sidx.at[nxt])

            pltpu.sync_copy(edge_hbm.at[1, pl.ds(row0 + g * NB, NB)], didx)
            for b in range(NB):
                pltpu.make_async_copy(h_hbm.at[sidx.at[par, b]], rows.at[b],
                                      gsem[b]).wait()
                pltpu.sync_copy(rows.at[b], acc.at[didx.at[b]], add=True)

                @pl.when(g < NG - 1)
                def _():
                    pltpu.async_copy(h_hbm.at[sidx.at[nxt, b]], rows.at[b],
                                     gsem[b])
            return carry

        lax.fori_loop(0, NG, group, 0)
        plsc.subcore_barrier()
        pltpu.sync_copy(acc.at[pl.ds(r0, INIT_ROWS)],
                        out_hbm.at[c, pl.ds(r0, INIT_ROWS)])

    return _scatter


_scatter128 = _make_scatter(HID, 2)
_scatter64 = _make_scatter(OUT_D, 8)


def _t1_body(deg_ref, x_ref, w_ref, h_ref, dinv_ref):
    deg = deg_ref[0] + deg_ref[1] + 1.0
    dinv = lax.rsqrt(deg)
    dr = jnp.broadcast_to(dinv[:, 0:1], (RB, HID))
    h = jnp.dot(x_ref[...], w_ref[...], preferred_element_type=jnp.float32)
    h_ref[...] = h * dr
    dinv_ref[...] = dr


def _t_mid_body(p_ref, h_ref, dinv_ref, b_ref, g_ref, be_ref, w_ref, o_ref):
    s = p_ref[0] + p_ref[1] - h_ref[...]
    o = s * dinv_ref[...] + b_ref[...]
    mu = jnp.mean(o, axis=1, keepdims=True)
    d = o - mu
    var = jnp.mean(d * d, axis=1, keepdims=True)
    ln = d * lax.rsqrt(var + EPS) * g_ref[...] + be_ref[...]
    r = jnp.maximum(ln, 0.0)
    o_ref[...] = (jnp.dot(r, w_ref[...], preferred_element_type=jnp.float32)
                  * dinv_ref[:, 0:1])


def _t4_body(p_ref, h_ref, dinv_ref, b_ref, o_ref):
    s = p_ref[0] + p_ref[1] - h_ref[...]
    o_ref[...] = s * dinv_ref[:, 0:1] + b_ref[...]


def _row_spec(d):
    return pl.BlockSpec((RB, d), lambda i: (i, 0))


def _pair_spec(d):
    return pl.BlockSpec((NCORES, RB, d), lambda i: (0, i, 0))


def _const_spec(shape):
    nd = len(shape)
    return pl.BlockSpec(shape, lambda i: (0,) * nd)


def kernel(x, edge_index, W1, b1, W2, b2, W3, b3, gamma, beta):
    src = edge_index[0].astype(jnp.int32)
    dst = edge_index[1].astype(jnp.int32)
    pad = PADE - E
    pad_src = (jnp.arange(pad, dtype=jnp.int32) * 37) % N
    pad_dst = N + (jnp.arange(pad, dtype=jnp.int32) % (NP - N))
    edge_p = jnp.concatenate(
        [jnp.stack([src, dst]), jnp.stack([pad_src, pad_dst])],
        axis=1).reshape(2, NCHUNK, C)
    ones_c = jnp.ones((C, DREP), jnp.float32)
    zeros_n = jnp.zeros((N, DREP), jnp.float32)
    b1r = b1.reshape(1, HID)
    b2r = b2.reshape(1, HID)
    b3r = b3.reshape(1, OUT_D)
    gr = gamma.reshape(1, HID)
    ber = beta.reshape(1, HID)

    deg_p = _deg_kernel(edge_p, ones_c, zeros_n)

    h1, dinv = pl.pallas_call(
        _t1_body,
        grid=(GRID,),
        in_specs=[_pair_spec(DREP), _row_spec(IN_D), _const_spec((IN_D, HID))],
        out_specs=[_row_spec(HID), _row_spec(HID)],
        out_shape=[jax.ShapeDtypeStruct((N, HID), jnp.float32),
                   jax.ShapeDtypeStruct((N, HID), jnp.float32)],
    )(deg_p, x, W1)

    p1 = _scatter128(h1, edge_p)

    h2 = pl.pallas_call(
        _t_mid_body,
        grid=(GRID,),
        in_specs=[_pair_spec(HID), _row_spec(HID), _row_spec(HID),
                  _const_spec((1, HID)), _const_spec((1, HID)),
                  _const_spec((1, HID)), _const_spec((HID, HID))],
        out_specs=_row_spec(HID),
        out_shape=jax.ShapeDtypeStruct((N, HID), jnp.float32),
    )(p1, h1, dinv, b1r, gr, ber, W2)

    p2 = _scatter128(h2, edge_p)

    h3 = pl.pallas_call(
        _t_mid_body,
        grid=(GRID,),
        in_specs=[_pair_spec(HID), _row_spec(HID), _row_spec(HID),
                  _const_spec((1, HID)), _const_spec((1, HID)),
                  _const_spec((1, HID)), _const_spec((HID, OUT_D))],
        out_specs=_row_spec(OUT_D),
        out_shape=jax.ShapeDtypeStruct((N, OUT_D), jnp.float32),
    )(p2, h2, dinv, b2r, gr, ber, W3)

    p3 = _scatter64(h3, edge_p)

    out = pl.pallas_call(
        _t4_body,
        grid=(GRID,),
        in_specs=[_pair_spec(OUT_D), _row_spec(OUT_D), _row_spec(HID),
                  _const_spec((1, OUT_D))],
        out_specs=_row_spec(OUT_D),
        out_shape=jax.ShapeDtypeStruct((N, OUT_D), jnp.float32),
    )(p3, h3, dinv, b3r)

    return out


# speedup vs baseline: 1.1862x; 1.0087x over previous
import functools

import jax
import jax.numpy as jnp
from jax import lax
from jax.experimental import pallas as pl
from jax.experimental.pallas import tpu as pltpu
from jax.experimental.pallas import tpu_sc as plsc

N = 10000
NP = 10240
E = 320000
IN_D = 128
HID = 128
OUT_D = 64

C = 128
NCORES = 2
NSUB = 16
NW = NCORES * NSUB
CPT = 80
PADE = NW * CPT * C
NCHUNK = PADE // C
INIT_ROWS = N // NSUB
DREP = 16
EPS = 1e-5

RB = 2000
GRID = N // RB


def _sc_mesh():
    return plsc.VectorSubcoreMesh(core_axis_name="c", subcore_axis_name="s")


@functools.partial(
    pl.kernel,
    out_type=jax.ShapeDtypeStruct((NCORES, N, DREP), jnp.float32),
    mesh=_sc_mesh(),
    compiler_params=pltpu.CompilerParams(use_tc_tiling_on_sc=False),
    scratch_types=[
        pltpu.VMEM((CPT, C), jnp.int32),
        pltpu.VMEM((C, DREP), jnp.float32),
        pltpu.VMEM_SHARED((NP, DREP), jnp.float32),
        pltpu.SemaphoreType.DMA,
    ],
)
def _deg_kernel(edge_hbm, ones_hbm, zero_hbm, out_hbm, idx, ones_v, acc,
                dsem):
    c = lax.axis_index("c")
    s = lax.axis_index("s")
    w = c * NSUB + s
    r0 = s * INIT_ROWS
    pltpu.sync_copy(zero_hbm.at[pl.ds(r0, INIT_ROWS)],
                    acc.at[pl.ds(r0, INIT_ROWS)])
    tr = (NP - N) // NSUB

    @pl.when(s == 0)
    def _():
        pltpu.sync_copy(zero_hbm.at[pl.ds(0, NP - N)],
                        acc.at[pl.ds(N, NP - N)])

    del tr
    pltpu.sync_copy(ones_hbm, ones_v)
    pltpu.sync_copy(edge_hbm.at[1, pl.ds(w * CPT, CPT)], idx)
    plsc.subcore_barrier()

    DB = 16

    def batch(i, carry):
        for b in range(DB):
            pltpu.async_copy(ones_v, acc.at[idx.at[i * DB + b]], dsem,
                             add=True)
        for b in range(DB):
            pltpu.make_async_copy(ones_v, acc.at[idx.at[i * DB + b]],
                                  dsem).wait()
        return carry

    lax.fori_loop(0, CPT // DB, batch, 0)
    plsc.subcore_barrier()
    pltpu.sync_copy(acc.at[pl.ds(r0, INIT_ROWS)],
                    out_hbm.at[c, pl.ds(r0, INIT_ROWS)])


def _make_scatter(D, NB):
    NG = CPT // NB

    @functools.partial(
        pl.kernel,
        out_type=jax.ShapeDtypeStruct((NCORES, N, D), jnp.float32),
        mesh=_sc_mesh(),
        compiler_params=pltpu.CompilerParams(use_tc_tiling_on_sc=False),
        scratch_types=[
            pltpu.VMEM((2, NB, C), jnp.int32),
            pltpu.VMEM((NB, C), jnp.int32),
            pltpu.VMEM((NB, C, D), jnp.float32),
            pltpu.VMEM_SHARED((NP, D), jnp.float32),
        ] + [pltpu.SemaphoreType.DMA] * NB,
    )
    def _scatter(h_hbm, edge_hbm, out_hbm, sidx, didx, rows, acc, *gsem):
        c = lax.axis_index("c")
        s = lax.axis_index("s")
        w = c * NSUB + s
        r0 = s * INIT_ROWS
        row0 = w * CPT
        pltpu.sync_copy(h_hbm.at[pl.ds(r0, INIT_ROWS)],
                        acc.at[pl.ds(r0, INIT_ROWS)])
        pltpu.sync_copy(edge_hbm.at[0, pl.ds(row0, NB)], sidx.at[0])
        plsc.subcore_barrier()
        for b in range(NB):
            pltpu.async_copy(h_hbm.at[sidx.at[0, b]], rows.at[b], gsem[b])

        def group(g, carry):
            par = lax.rem(g, 2)
            nxt = 1 - par

            @pl.when(g < NG - 1)
            def _():
                pltpu.sync_copy(edge_hbm.at[0, pl.ds(row0 + (g + 1) * NB, NB)],
                                sidx.at[nxt])

            pltpu.sync_copy(edge_hbm.at[1, pl.ds(row0 + g * NB, NB)], didx)
            for b in range(NB):
                pltpu.make_async_copy(h_hbm.at[sidx.at[par, b]], rows.at[b],
                                      gsem[b]).wait()
                pltpu.sync_copy(rows.at[b], acc.at[didx.at[b]], add=True)

                @pl.when(g < NG - 1)
                def _():
                    pltpu.async_copy(h_hbm.at[sidx.at[nxt, b]], rows.at[b],
                                     gsem[b])
            return carry

        lax.fori_loop(0, NG, group, 0)
        plsc.subcore_barrier()
        pltpu.sync_copy(acc.at[pl.ds(r0, INIT_ROWS)],
                        out_hbm.at[c, pl.ds(r0, INIT_ROWS)])

    return _scatter


_scatter128 = _make_scatter(HID, 2)
_scatter64 = _make_scatter(OUT_D, 8)


def _t1_body(deg_ref, x_ref, w_ref, h_ref, dinv_ref):
    deg = deg_ref[0] + deg_ref[1] + 1.0
    dinv = lax.rsqrt(deg)
    dr = jnp.broadcast_to(dinv[:, 0:1], (RB, HID))
    h = jnp.dot(x_ref[...], w_ref[...], preferred_element_type=jnp.float32)
    h_ref[...] = h * dr
    dinv_ref[...] = dr


def _t_mid_body(p_ref, h_ref, dinv_ref, b_ref, g_ref, be_ref, w_ref, o_ref):
    s = p_ref[0] + p_ref[1] - h_ref[...]
    o = s * dinv_ref[...] + b_ref[...]
    mu = jnp.mean(o, axis=1, keepdims=True)
    d = o - mu
    var = jnp.mean(d * d, axis=1, keepdims=True)
    ln = d * lax.rsqrt(var + EPS) * g_ref[...] + be_ref[...]
    r = jnp.maximum(ln, 0.0)
    o_ref[...] = (jnp.dot(r, w_ref[...], preferred_element_type=jnp.float32)
                  * dinv_ref[:, 0:1])


def _t4_body(p_ref, h_ref, dinv_ref, b_ref, o_ref):
    s = p_ref[0] + p_ref[1] - h_ref[...]
    o_ref[...] = s * dinv_ref[:, 0:1] + b_ref[...]


def _row_spec(d):
    return pl.BlockSpec((RB, d), lambda i: (i, 0))


def _pair_spec(d):
    return pl.BlockSpec((NCORES, RB, d), lambda i: (0, i, 0))


def _const_spec(shape):
    nd = len(shape)
    return pl.BlockSpec(shape, lambda i: (0,) * nd)


def kernel(x, edge_index, W1, b1, W2, b2, W3, b3, gamma, beta):
    src = edge_index[0].astype(jnp.int32)
    dst = edge_index[1].astype(jnp.int32)
    pad = PADE - E
    pad_src = (jnp.arange(pad, dtype=jnp.int32) * 37) % N
    pad_dst = N + (jnp.arange(pad, dtype=jnp.int32) % (NP - N))
    edge_p = jnp.concatenate(
        [jnp.stack([src, dst]), jnp.stack([pad_src, pad_dst])],
        axis=1).reshape(2, NCHUNK, C)
    ones_c = jnp.ones((C, DREP), jnp.float32)
    zeros_n = jnp.zeros((N, DREP), jnp.float32)
    b1r = b1.reshape(1, HID)
    b2r = b2.reshape(1, HID)
    b3r = b3.reshape(1, OUT_D)
    gr = gamma.reshape(1, HID)
    ber = beta.reshape(1, HID)

    deg_p = _deg_kernel(edge_p, ones_c, zeros_n)

    h1, dinv = pl.pallas_call(
        _t1_body,
        grid=(GRID,),
        in_specs=[_pair_spec(DREP), _row_spec(IN_D), _const_spec((IN_D, HID))],
        out_specs=[_row_spec(HID), _row_spec(HID)],
        out_shape=[jax.ShapeDtypeStruct((N, HID), jnp.float32),
                   jax.ShapeDtypeStruct((N, HID), jnp.float32)],
    )(deg_p, x, W1)

    p1 = _scatter128(h1, edge_p)

    h2 = pl.pallas_call(
        _t_mid_body,
        grid=(GRID,),
        in_specs=[_pair_spec(HID), _row_spec(HID), _row_spec(HID),
                  _const_spec((1, HID)), _const_spec((1, HID)),
                  _const_spec((1, HID)), _const_spec((HID, HID))],
        out_specs=_row_spec(HID),
        out_shape=jax.ShapeDtypeStruct((N, HID), jnp.float32),
    )(p1, h1, dinv, b1r, gr, ber, W2)

    p2 = _scatter128(h2, edge_p)

    h3 = pl.pallas_call(
        _t_mid_body,
        grid=(GRID,),
        in_specs=[_pair_spec(HID), _row_spec(HID), _row_spec(HID),
                  _const_spec((1, HID)), _const_spec((1, HID)),
                  _const_spec((1, HID)), _const_spec((HID, OUT_D))],
        out_specs=_row_spec(OUT_D),
        out_shape=jax.ShapeDtypeStruct((N, OUT_D), jnp.float32),
    )(p2, h2, dinv, b2r, gr, ber, W3)

    p3 = _scatter64(h3, edge_p)

    out = pl.pallas_call(
        _t4_body,
        grid=(GRID,),
        in_specs=[_pair_spec(OUT_D), _row_spec(OUT_D), _row_spec(HID),
                  _const_spec((1, OUT_D))],
        out_specs=_row_spec(OUT_D),
        out_shape=jax.ShapeDtypeStruct((N, OUT_D), jnp.float32),
    )(p3, h3, dinv, b3r)

    return out
